# baseline (speedup 1.0000x reference)
import functools

import jax
import jax.numpy as jnp
from jax import lax
from jax.experimental import pallas as pl
from jax.experimental.pallas import tpu as pltpu
from jax.experimental.pallas import tpu_sc as plsc

N = 10000
E = 320000
CH = 128
NC, NS, L = 2, 16, 16
NW = NC * NS
EPW = E // NW
CHUNK = 80
NCHUNK = EPW // CHUNK
NPAD = 10240
ROWS_PT = NPAD // NS
DEN_PAD = 10240
DEN_PT = DEN_PAD // NS
SEG = 25
NSEG = NCHUNK // SEG

_mesh = plsc.VectorSubcoreMesh(
    core_axis_name="c", subcore_axis_name="s", num_cores=NC, num_subcores=NS
)


def _proj_body(x_ref, w_ref, asv_ref, adv_ref,
               xp_ref, as_ref, ad_ref, c_ref, ms_ref, md_ref):
    i = pl.program_id(0)
    xp = jnp.dot(x_ref[...], w_ref[...], preferred_element_type=jnp.float32)
    xp_ref[...] = xp
    s = jnp.sum(xp * asv_ref[...], axis=1, keepdims=True)
    d = jnp.sum(xp * adv_ref[...], axis=1, keepdims=True)
    as_ref[...] = s
    ad_ref[...] = d

    @pl.when(i == 0)
    def _():
        ms_ref[0] = -jnp.inf
        md_ref[0] = -jnp.inf

    ms_ref[0] = jnp.maximum(ms_ref[0], jnp.max(s))
    md_ref[0] = jnp.maximum(md_ref[0], jnp.max(d))

    @pl.when(i == pl.num_programs(0) - 1)
    def _():
        m = ms_ref[0] + md_ref[0]
        c_ref[...] = jnp.broadcast_to(jnp.where(m >= 0.0, m, 0.2 * m), (1, 1))


_PB = 400


def _project(x, W, a_src_v, a_dst_v):
    return pl.pallas_call(
        _proj_body,
        grid=(N // _PB,),
        in_specs=[
            pl.BlockSpec((_PB, CH), lambda i: (i, 0)),
            pl.BlockSpec((CH, CH), lambda i: (0, 0)),
            pl.BlockSpec((1, CH), lambda i: (0, 0)),
            pl.BlockSpec((1, CH), lambda i: (0, 0)),
        ],
        out_specs=[
            pl.BlockSpec((_PB, CH), lambda i: (i, 0)),
            pl.BlockSpec((_PB, 1), lambda i: (i, 0)),
            pl.BlockSpec((_PB, 1), lambda i: (i, 0)),
            pl.BlockSpec((1, 1), lambda i: (0, 0)),
        ],
        out_shape=[
            jax.ShapeDtypeStruct((N, CH), jnp.float32),
            jax.ShapeDtypeStruct((N, 1), jnp.float32),
            jax.ShapeDtypeStruct((N, 1), jnp.float32),
            jax.ShapeDtypeStruct((1, 1), jnp.float32),
        ],
        scratch_shapes=[
            pltpu.SMEM((1,), jnp.float32),
            pltpu.SMEM((1,), jnp.float32),
        ],
    )(x, W, a_src_v, a_dst_v)


@functools.partial(
    pl.kernel,
    out_type=(
        jax.ShapeDtypeStruct((NC, NPAD, CH), jnp.float32),
        jax.ShapeDtypeStruct((NW, DEN_PAD), jnp.float32),
    ),
    mesh=_mesh,
    scratch_types=[
        pltpu.VMEM((N,), jnp.float32),
        pltpu.VMEM((N,), jnp.float32),
        pltpu.VMEM((L,), jnp.float32),
        pltpu.VMEM((SEG, CHUNK), jnp.int32),
        pltpu.VMEM((SEG, CHUNK), jnp.int32),
        pltpu.VMEM((CHUNK,), jnp.float32),
        pltpu.VMEM((CHUNK, CH), jnp.float32),
        pltpu.VMEM((DEN_PAD,), jnp.float32),
        pltpu.VMEM_SHARED((NPAD, CH), jnp.float32),
        pltpu.SemaphoreType.DMA,
    ],
    compiler_params=pltpu.CompilerParams(needs_layout_passes=False),
)
def _sc_edges(xp_hbm, asrc_hbm, adst_hbm, src_hbm, dst_hbm, cvec_hbm,
              outp_hbm, denp_hbm,
              asrc_v, adst_v, cvec_v, src2_v, dst2_v, ex_v, rows_v,
              den_pv, out_sh, sem):
    cid = lax.axis_index("c")
    sid = lax.axis_index("s")
    wid = cid * NS + sid

    zf = jnp.zeros((L,), jnp.float32)

    @pl.loop(0, DEN_PAD, step=L)
    def _(j):
        den_pv[pl.ds(j, L)] = zf

    @pl.loop(0, CHUNK)
    def _(r):
        for j in range(CH // L):
            rows_v[r, pl.ds(j * L, L)] = zf

    @pl.loop(0, ROWS_PT // CHUNK)
    def _(k):
        pltpu.sync_copy(rows_v,
                        out_sh.at[pl.ds(sid * ROWS_PT + k * CHUNK, CHUNK)])

    pltpu.sync_copy(asrc_hbm, asrc_v)
    pltpu.sync_copy(adst_hbm, adst_v)
    pltpu.sync_copy(cvec_hbm, cvec_v)

    plsc.subcore_barrier()

    cvec = cvec_v[...]

    @pl.loop(0, NSEG)
    def _(sg):
        pltpu.sync_copy(src_hbm.at[wid, sg], src2_v)
        pltpu.sync_copy(dst_hbm.at[wid, sg], dst2_v)

        @pl.loop(0, SEG)
        def _(t):
            gather = pltpu.async_copy(xp_hbm.at[src2_v.at[t]], rows_v, sem)

            @pl.loop(0, CHUNK, step=L)
            def _(j):
                sv = src2_v[t, pl.ds(j, L)]
                dv = dst2_v[t, pl.ds(j, L)]
                a = (plsc.load_gather(asrc_v, [sv])
                     + plsc.load_gather(adst_v, [dv]))
                a = jnp.where(a >= 0.0, a, 0.2 * a)
                e = jnp.exp(a - cvec)
                ex_v[pl.ds(j, L)] = e
                plsc.addupdate_scatter(den_pv, [dv], e)

            gather.wait()

            @pl.loop(0, CHUNK)
            def _(e):
                s = plsc.load_gather(ex_v, [jnp.zeros((L,), jnp.int32) + e])
                for j in range(CH // L):
                    rows_v[e, pl.ds(j * L, L)] = rows_v[e, pl.ds(j * L, L)] * s

            pltpu.sync_copy(rows_v, out_sh.at[dst2_v.at[t]], add=True)

    plsc.subcore_barrier()

    r0 = sid * ROWS_PT
    pltpu.sync_copy(out_sh.at[pl.ds(r0, ROWS_PT)],
                    outp_hbm.at[cid, pl.ds(r0, ROWS_PT)])
    pltpu.sync_copy(den_pv, denp_hbm.at[wid])


def _fin_body(o_ref, d_ref, b_ref, p_ref, out_ref):
    o = o_ref[0] + o_ref[1]
    den = jnp.sum(d_ref[...], axis=0)
    r = o / (den + 1e-16) + b_ref[...]
    out_ref[...] = jnp.where(r >= 0.0, r, p_ref[...] * r)


_FB = 400


def _finalize(outp, denp, bias2, prelu2):
    return pl.pallas_call(
        _fin_body,
        grid=(N // _FB,),
        in_specs=[
            pl.BlockSpec((NC, _FB, CH), lambda i: (0, i, 0)),
            pl.BlockSpec((NW, _FB, 1), lambda i: (0, i, 0)),

            pl.BlockSpec((1, CH), lambda i: (0, 0)),
            pl.BlockSpec((1, CH), lambda i: (0, 0)),
        ],
        out_specs=pl.BlockSpec((_FB, CH), lambda i: (i, 0)),
        out_shape=jax.ShapeDtypeStruct((N, CH), jnp.float32),
    )(outp, denp, bias2, prelu2)


def kernel(x, edge_index, W, a_src, a_dst, bias, prelu_w):
    src = edge_index[0].astype(jnp.int32)
    dst = edge_index[1].astype(jnp.int32)
    a_src_v = a_src.reshape(1, CH).astype(jnp.float32)
    a_dst_v = a_dst.reshape(1, CH).astype(jnp.float32)

    xp, asrc, adst, cmax = _project(x, W, a_src_v, a_dst_v)
    cvec = jnp.broadcast_to(cmax[0, 0], (L,))

    src3 = src.reshape(NW, NSEG, SEG, CHUNK)
    dst3 = dst.reshape(NW, NSEG, SEG, CHUNK)
    outp, denp = _sc_edges(xp, asrc[:, 0], adst[:, 0], src3, dst3, cvec)

    den = denp.reshape(NW, DEN_PAD, 1)
    out = _finalize(outp, den, bias.reshape(1, CH), prelu_w.reshape(1, CH))
    return out


# speedup vs baseline: 1.4307x; 1.4307x over previous
import functools

import jax
import jax.numpy as jnp
from jax import lax
from jax.experimental import pallas as pl
from jax.experimental.pallas import tpu as pltpu
from jax.experimental.pallas import tpu_sc as plsc

N = 10000
E = 320000
CH = 128
NC, NS, L = 2, 16, 16
NW = NC * NS
EPW = E // NW
CHUNK = 80
NCHUNK = EPW // CHUNK
NPAD = 10240
ROWS_PT = NPAD // NS
DEN_PAD = 10240
DEN_PT = DEN_PAD // NS
SEG = 25
NSEG = NCHUNK // SEG

_GDN = lax.GatherDimensionNumbers(
    offset_dims=(), collapsed_slice_dims=(0,), start_index_map=(0,))


def _bcast(vec, k):
    idx = jnp.full((L, 1), k, jnp.int32)
    return lax.gather(vec, idx, _GDN, (1,),
                      mode=lax.GatherScatterMode.PROMISE_IN_BOUNDS)


_mesh = plsc.VectorSubcoreMesh(
    core_axis_name="c", subcore_axis_name="s", num_cores=NC, num_subcores=NS
)


def _proj_body(x_ref, w_ref, asv_ref, adv_ref,
               xp_ref, as_ref, ad_ref, c_ref, ms_ref, md_ref):
    i = pl.program_id(0)
    xp = jnp.dot(x_ref[...], w_ref[...], preferred_element_type=jnp.float32)
    xp_ref[...] = xp
    s = jnp.sum(xp * asv_ref[...], axis=1, keepdims=True)
    d = jnp.sum(xp * adv_ref[...], axis=1, keepdims=True)
    as_ref[...] = s
    ad_ref[...] = d

    @pl.when(i == 0)
    def _():
        ms_ref[0] = -jnp.inf
        md_ref[0] = -jnp.inf

    ms_ref[0] = jnp.maximum(ms_ref[0], jnp.max(s))
    md_ref[0] = jnp.maximum(md_ref[0], jnp.max(d))

    @pl.when(i == pl.num_programs(0) - 1)
    def _():
        m = ms_ref[0] + md_ref[0]
        c_ref[...] = jnp.broadcast_to(jnp.where(m >= 0.0, m, 0.2 * m), (1, 1))


_PB = 400


def _project(x, W, a_src_v, a_dst_v):
    return pl.pallas_call(
        _proj_body,
        grid=(N // _PB,),
        in_specs=[
            pl.BlockSpec((_PB, CH), lambda i: (i, 0)),
            pl.BlockSpec((CH, CH), lambda i: (0, 0)),
            pl.BlockSpec((1, CH), lambda i: (0, 0)),
            pl.BlockSpec((1, CH), lambda i: (0, 0)),
        ],
        out_specs=[
            pl.BlockSpec((_PB, CH), lambda i: (i, 0)),
            pl.BlockSpec((_PB, 1), lambda i: (i, 0)),
            pl.BlockSpec((_PB, 1), lambda i: (i, 0)),
            pl.BlockSpec((1, 1), lambda i: (0, 0)),
        ],
        out_shape=[
            jax.ShapeDtypeStruct((N, CH), jnp.float32),
            jax.ShapeDtypeStruct((N, 1), jnp.float32),
            jax.ShapeDtypeStruct((N, 1), jnp.float32),
            jax.ShapeDtypeStruct((1, 1), jnp.float32),
        ],
        scratch_shapes=[
            pltpu.SMEM((1,), jnp.float32),
            pltpu.SMEM((1,), jnp.float32),
        ],
    )(x, W, a_src_v, a_dst_v)


@functools.partial(
    pl.kernel,
    out_type=(
        jax.ShapeDtypeStruct((NC, NPAD, CH), jnp.float32),
        jax.ShapeDtypeStruct((NW, DEN_PAD), jnp.float32),
    ),
    mesh=_mesh,
    scratch_types=[
        pltpu.VMEM((L,), jnp.float32),
        pltpu.VMEM((SEG, CHUNK), jnp.int32),
        pltpu.VMEM((SEG, CHUNK), jnp.int32),
        pltpu.VMEM((SEG * CHUNK,), jnp.float32),
        pltpu.VMEM((CHUNK, CH), jnp.float32),
        pltpu.VMEM((CHUNK, CH), jnp.float32),
        pltpu.VMEM((CHUNK,), jnp.float32),
        pltpu.VMEM((CHUNK,), jnp.float32),
        pltpu.VMEM((CHUNK,), jnp.float32),
        pltpu.VMEM((CHUNK,), jnp.float32),
        pltpu.VMEM((DEN_PAD,), jnp.float32),
        pltpu.VMEM_SHARED((NPAD, CH), jnp.float32),
        pltpu.SemaphoreType.DMA,
        pltpu.SemaphoreType.DMA,
        pltpu.SemaphoreType.DMA,
        pltpu.SemaphoreType.DMA,
        pltpu.SemaphoreType.DMA,
        pltpu.SemaphoreType.DMA,
        pltpu.SemaphoreType.DMA,
        pltpu.SemaphoreType.DMA,
    ],
    compiler_params=pltpu.CompilerParams(needs_layout_passes=False),
)
def _sc_edges(xp_hbm, asrc_hbm, adst_hbm, src_hbm, dst_hbm, cvec_hbm,
              outp_hbm, denp_hbm,
              cvec_v, src2_v, dst2_v, ex_v, rows_a, rows_b,
              av_a, av_b, bv_a, bv_b,
              den_pv, out_sh, gsem_a, gsem_b,
              asem_a, asem_b, bsem_a, bsem_b, osem_a, osem_b):
    cid = lax.axis_index("c")
    sid = lax.axis_index("s")
    wid = cid * NS + sid

    zf = jnp.zeros((L,), jnp.float32)

    @pl.loop(0, DEN_PAD, step=L)
    def _(j):
        den_pv[pl.ds(j, L)] = zf

    @pl.loop(0, CHUNK)
    def _(r):
        for j in range(CH // L):
            rows_a[r, pl.ds(j * L, L)] = zf

    @pl.loop(0, ROWS_PT // CHUNK)
    def _(k):
        pltpu.sync_copy(rows_a,
                        out_sh.at[pl.ds(sid * ROWS_PT + k * CHUNK, CHUNK)])

    pltpu.sync_copy(cvec_hbm, cvec_v)

    plsc.subcore_barrier()

    cvec = cvec_v[...]

    def prefetch(t, rows, av, bv, gsem, asem, bsem):
        pltpu.async_copy(xp_hbm.at[src2_v.at[t]], rows, gsem)
        pltpu.async_copy(asrc_hbm.at[src2_v.at[t]], av, asem)
        pltpu.async_copy(adst_hbm.at[dst2_v.at[t]], bv, bsem)

    def half(t, rows_x, av_x, bv_x, gsem_x, asem_x, bsem_x, osem_x,
             rows_y, av_y, bv_y, gsem_y, asem_y, bsem_y, osem_y):
        eb = t * CHUNK
        pltpu.make_async_copy(asrc_hbm.at[src2_v.at[t]], av_x, asem_x).wait()
        pltpu.make_async_copy(adst_hbm.at[dst2_v.at[t]], bv_x, bsem_x).wait()
        for j in range(0, CHUNK, L):
            a = av_x[pl.ds(j, L)] + bv_x[pl.ds(j, L)]
            a = jnp.where(a >= 0.0, a, 0.2 * a)
            e = jnp.exp(a - cvec)
            ex_v[pl.ds(eb + j, L)] = e
            plsc.addupdate_scatter(den_pv, [dst2_v[t, pl.ds(j, L)]], e)

        @pl.when(jnp.asarray(t + 1 < SEG))
        def _():
            @pl.when(jnp.asarray(t >= 1))
            def _():
                pltpu.make_async_copy(
                    rows_y, out_sh.at[dst2_v.at[t]], osem_y).wait()
            prefetch(t + 1, rows_y, av_y, bv_y, gsem_y, asem_y, bsem_y)

        pltpu.make_async_copy(xp_hbm.at[src2_v.at[t]], rows_x, gsem_x).wait()

        @pl.loop(0, CHUNK, step=L)
        def _(j):
            evec = ex_v[pl.ds(eb + j, L)]
            for k in range(L):
                s = _bcast(evec, k)
                for c in range(CH // L):
                    rows_x[j + k, pl.ds(c * L, L)] = (
                        rows_x[j + k, pl.ds(c * L, L)] * s)

        pltpu.async_copy(rows_x, out_sh.at[dst2_v.at[t]], osem_x, add=True)

    @pl.loop(0, NSEG)
    def _(sg):
        pltpu.sync_copy(src_hbm.at[wid, sg], src2_v)
        pltpu.sync_copy(dst_hbm.at[wid, sg], dst2_v)
        prefetch(0, rows_a, av_a, bv_a, gsem_a, asem_a, bsem_a)

        @pl.loop(0, SEG // 2)
        def _(tt):
            half(2 * tt, rows_a, av_a, bv_a, gsem_a, asem_a, bsem_a, osem_a,
                 rows_b, av_b, bv_b, gsem_b, asem_b, bsem_b, osem_b)
            half(2 * tt + 1,
                 rows_b, av_b, bv_b, gsem_b, asem_b, bsem_b, osem_b,
                 rows_a, av_a, bv_a, gsem_a, asem_a, bsem_a, osem_a)

        half(SEG - 1, rows_a, av_a, bv_a, gsem_a, asem_a, bsem_a, osem_a,
             rows_b, av_b, bv_b, gsem_b, asem_b, bsem_b, osem_b)

        pltpu.make_async_copy(rows_b, out_sh.at[dst2_v.at[0]], osem_b).wait()
        pltpu.make_async_copy(rows_a, out_sh.at[dst2_v.at[0]], osem_a).wait()

    plsc.subcore_barrier()

    r0 = sid * ROWS_PT
    pltpu.sync_copy(out_sh.at[pl.ds(r0, ROWS_PT)],
                    outp_hbm.at[cid, pl.ds(r0, ROWS_PT)])
    pltpu.sync_copy(den_pv, denp_hbm.at[wid])


def _fin_body(o_ref, d_ref, b_ref, p_ref, out_ref):
    o = o_ref[0] + o_ref[1]
    den = jnp.sum(d_ref[...], axis=0)
    r = o / (den + 1e-16) + b_ref[...]
    out_ref[...] = jnp.where(r >= 0.0, r, p_ref[...] * r)


_FB = 400


def _finalize(outp, denp, bias2, prelu2):
    return pl.pallas_call(
        _fin_body,
        grid=(N // _FB,),
        in_specs=[
            pl.BlockSpec((NC, _FB, CH), lambda i: (0, i, 0)),
            pl.BlockSpec((NW, _FB, 1), lambda i: (0, i, 0)),

            pl.BlockSpec((1, CH), lambda i: (0, 0)),
            pl.BlockSpec((1, CH), lambda i: (0, 0)),
        ],
        out_specs=pl.BlockSpec((_FB, CH), lambda i: (i, 0)),
        out_shape=jax.ShapeDtypeStruct((N, CH), jnp.float32),
    )(outp, denp, bias2, prelu2)


def kernel(x, edge_index, W, a_src, a_dst, bias, prelu_w):
    src = edge_index[0].astype(jnp.int32)
    dst = edge_index[1].astype(jnp.int32)
    a_src_v = a_src.reshape(1, CH).astype(jnp.float32)
    a_dst_v = a_dst.reshape(1, CH).astype(jnp.float32)

    xp, asrc, adst, cmax = _project(x, W, a_src_v, a_dst_v)
    cvec = jnp.broadcast_to(cmax[0, 0], (L,))

    src3 = src.reshape(NW, NSEG, SEG, CHUNK)
    dst3 = dst.reshape(NW, NSEG, SEG, CHUNK)
    outp, denp = _sc_edges(xp, asrc[:, 0], adst[:, 0], src3, dst3, cvec)

    den = denp.reshape(NW, DEN_PAD, 1)
    out = _finalize(outp, den, bias.reshape(1, CH), prelu_w.reshape(1, CH))
    return out


# speedup vs baseline: 2.2294x; 1.5583x over previous
import functools

import jax
import jax.numpy as jnp
from jax import lax
from jax.experimental import pallas as pl
from jax.experimental.pallas import tpu as pltpu
from jax.experimental.pallas import tpu_sc as plsc

N = 10000
E = 320000
CH = 128
NC, NS, L = 2, 16, 16
NW = NC * NS
EPW = E // NW
CHUNK = 80
NCHUNK = EPW // CHUNK
NPAD = 10240
ROWS_PT = NPAD // NS
DEN_PAD = 10240
DEN_PT = DEN_PAD // NS
SEG = 25
NSEG = NCHUNK // SEG

_GDN = lax.GatherDimensionNumbers(
    offset_dims=(), collapsed_slice_dims=(0,), start_index_map=(0,))


def _bcast(vec, k):
    idx = jnp.full((L, 1), k, jnp.int32)
    return lax.gather(vec, idx, _GDN, (1,),
                      mode=lax.GatherScatterMode.PROMISE_IN_BOUNDS)


_mesh = plsc.VectorSubcoreMesh(
    core_axis_name="c", subcore_axis_name="s", num_cores=NC, num_subcores=NS
)


_PB = 1024
_PR = _PB // 128


def _proj_body(x_ref, w_ref, asv_ref, adv_ref,
               xp_ref, as_ref, ad_ref, c_ref, ms_ref, md_ref):
    i = pl.program_id(0)
    xp = jnp.dot(x_ref[...], w_ref[...], preferred_element_type=jnp.float32)
    xp_ref[...] = xp
    rows_s = []
    rows_d = []
    for k in range(_PR):
        blk = xp[128 * k:128 * (k + 1), :]
        rows_s.append(jax.lax.dot_general(
            asv_ref[...], blk, (((1,), (1,)), ((), ())),
            preferred_element_type=jnp.float32))
        rows_d.append(jax.lax.dot_general(
            adv_ref[...], blk, (((1,), (1,)), ((), ())),
            preferred_element_type=jnp.float32))
    s = jnp.concatenate(rows_s, axis=0)
    d = jnp.concatenate(rows_d, axis=0)
    as_ref[...] = s
    ad_ref[...] = d

    nid = (i * _PB + 128 * jax.lax.broadcasted_iota(jnp.int32, (_PR, CH), 0)
           + jax.lax.broadcasted_iota(jnp.int32, (_PR, CH), 1))
    valid = nid < N
    neg = jnp.float32(-jnp.inf)

    @pl.when(i == 0)
    def _():
        ms_ref[0] = neg
        md_ref[0] = neg

    ms_ref[0] = jnp.maximum(ms_ref[0], jnp.max(jnp.where(valid, s, neg)))
    md_ref[0] = jnp.maximum(md_ref[0], jnp.max(jnp.where(valid, d, neg)))

    @pl.when(i == pl.num_programs(0) - 1)
    def _():
        m = ms_ref[0] + md_ref[0]
        c_ref[...] = jnp.broadcast_to(jnp.where(m >= 0.0, m, 0.2 * m), (1, 1))


def _project(xpad, W, a_src_v, a_dst_v):
    return pl.pallas_call(
        _proj_body,
        grid=(NPAD // _PB,),
        in_specs=[
            pl.BlockSpec((_PB, CH), lambda i: (i, 0)),
            pl.BlockSpec((CH, CH), lambda i: (0, 0)),
            pl.BlockSpec((1, CH), lambda i: (0, 0)),
            pl.BlockSpec((1, CH), lambda i: (0, 0)),
        ],
        out_specs=[
            pl.BlockSpec((_PB, CH), lambda i: (i, 0)),
            pl.BlockSpec((_PR, CH), lambda i: (i, 0)),
            pl.BlockSpec((_PR, CH), lambda i: (i, 0)),
            pl.BlockSpec((1, 1), lambda i: (0, 0)),
        ],
        out_shape=[
            jax.ShapeDtypeStruct((NPAD, CH), jnp.float32),
            jax.ShapeDtypeStruct((NPAD // CH, CH), jnp.float32),
            jax.ShapeDtypeStruct((NPAD // CH, CH), jnp.float32),
            jax.ShapeDtypeStruct((1, 1), jnp.float32),
        ],
        scratch_shapes=[
            pltpu.SMEM((1,), jnp.float32),
            pltpu.SMEM((1,), jnp.float32),
        ],
    )(xpad, W, a_src_v, a_dst_v)


@functools.partial(
    pl.kernel,
    out_type=(
        jax.ShapeDtypeStruct((NC, NPAD, CH), jnp.float32),
        jax.ShapeDtypeStruct((NC, DEN_PAD), jnp.float32),
    ),
    mesh=_mesh,
    scratch_types=[
        pltpu.VMEM((L,), jnp.float32),
        pltpu.VMEM((SEG, CHUNK), jnp.int32),
        pltpu.VMEM((SEG, CHUNK), jnp.int32),
        pltpu.VMEM((SEG * CHUNK,), jnp.float32),
        pltpu.VMEM((CHUNK, CH), jnp.float32),
        pltpu.VMEM((CHUNK, CH), jnp.float32),
        pltpu.VMEM((CHUNK,), jnp.float32),
        pltpu.VMEM((CHUNK,), jnp.float32),
        pltpu.VMEM((CHUNK,), jnp.float32),
        pltpu.VMEM((CHUNK,), jnp.float32),
        pltpu.VMEM_SHARED((NPAD, CH), jnp.float32),
        pltpu.VMEM_SHARED((DEN_PAD,), jnp.float32),
        pltpu.SemaphoreType.DMA,
        pltpu.SemaphoreType.DMA,
        pltpu.SemaphoreType.DMA,
        pltpu.SemaphoreType.DMA,
        pltpu.SemaphoreType.DMA,
        pltpu.SemaphoreType.DMA,
        pltpu.SemaphoreType.DMA,
        pltpu.SemaphoreType.DMA,
        pltpu.SemaphoreType.DMA,
        pltpu.SemaphoreType.DMA,
    ],
    compiler_params=pltpu.CompilerParams(needs_layout_passes=False),
)
def _sc_edges(xp_hbm, asrc_hbm, adst_hbm, src_hbm, dst_hbm, cvec_hbm,
              outp_hbm, denp_hbm,
              cvec_v, src2_v, dst2_v, ex_v, rows_a, rows_b,
              av_a, av_b, bv_a, bv_b,
              out_sh, den_sh, gsem_a, gsem_b,
              asem_a, asem_b, bsem_a, bsem_b, osem_a, osem_b,
              dsem_a, dsem_b):
    cid = lax.axis_index("c")
    sid = lax.axis_index("s")
    wid = cid * NS + sid

    zf = jnp.zeros((L,), jnp.float32)

    @pl.loop(0, DEN_PT, step=L)
    def _(j):
        ex_v[pl.ds(j, L)] = zf

    @pl.loop(0, CHUNK)
    def _(r):
        for j in range(CH // L):
            rows_a[r, pl.ds(j * L, L)] = zf

    @pl.loop(0, ROWS_PT // CHUNK)
    def _(k):
        pltpu.sync_copy(rows_a,
                        out_sh.at[pl.ds(sid * ROWS_PT + k * CHUNK, CHUNK)])

    pltpu.sync_copy(ex_v.at[pl.ds(0, DEN_PT)],
                    den_sh.at[pl.ds(sid * DEN_PT, DEN_PT)])
    pltpu.sync_copy(cvec_hbm, cvec_v)

    plsc.subcore_barrier()

    cvec = cvec_v[...]

    def prefetch(t, rows, av, bv, gsem, asem, bsem):
        pltpu.async_copy(xp_hbm.at[src2_v.at[t]], rows, gsem)
        pltpu.async_copy(asrc_hbm.at[src2_v.at[t]], av, asem)
        pltpu.async_copy(adst_hbm.at[dst2_v.at[t]], bv, bsem)

    def half(t, rows_x, av_x, bv_x, gsem_x, asem_x, bsem_x, osem_x,
             dsem_x, rows_y, av_y, bv_y, gsem_y, asem_y, bsem_y, osem_y):
        eb = t * CHUNK
        pltpu.make_async_copy(asrc_hbm.at[src2_v.at[t]], av_x, asem_x).wait()
        pltpu.make_async_copy(adst_hbm.at[dst2_v.at[t]], bv_x, bsem_x).wait()

        @pl.when(jnp.asarray(t >= 2))
        def _():
            pltpu.make_async_copy(ex_v.at[pl.ds(0, CHUNK)],
                                  den_sh.at[dst2_v.at[t]], dsem_x).wait()

        for j in range(0, CHUNK, L):
            a = av_x[pl.ds(j, L)] + bv_x[pl.ds(j, L)]
            a = jnp.where(a >= 0.0, a, 0.2 * a)
            ex_v[pl.ds(eb + j, L)] = jnp.exp(a - cvec)
        pltpu.async_copy(ex_v.at[pl.ds(eb, CHUNK)], den_sh.at[dst2_v.at[t]],
                         dsem_x, add=True)

        @pl.when(jnp.asarray(t + 1 < SEG))
        def _():
            @pl.when(jnp.asarray(t >= 1))
            def _():
                pltpu.make_async_copy(
                    rows_y, out_sh.at[dst2_v.at[t]], osem_y).wait()
            prefetch(t + 1, rows_y, av_y, bv_y, gsem_y, asem_y, bsem_y)

        pltpu.make_async_copy(xp_hbm.at[src2_v.at[t]], rows_x, gsem_x).wait()

        @pl.loop(0, CHUNK, step=L)
        def _(j):
            evec = ex_v[pl.ds(eb + j, L)]
            for k in range(L):
                s = _bcast(evec, k)
                for c in range(CH // L):
                    rows_x[j + k, pl.ds(c * L, L)] = (
                        rows_x[j + k, pl.ds(c * L, L)] * s)

        pltpu.async_copy(rows_x, out_sh.at[dst2_v.at[t]], osem_x, add=True)

    @pl.loop(0, NSEG)
    def _(sg):
        pltpu.sync_copy(src_hbm.at[wid, sg], src2_v)
        pltpu.sync_copy(dst_hbm.at[wid, sg], dst2_v)
        prefetch(0, rows_a, av_a, bv_a, gsem_a, asem_a, bsem_a)

        @pl.loop(0, SEG // 2)
        def _(tt):
            half(2 * tt, rows_a, av_a, bv_a,
                 gsem_a, asem_a, bsem_a, osem_a, dsem_a,
                 rows_b, av_b, bv_b, gsem_b, asem_b, bsem_b, osem_b)
            half(2 * tt + 1, rows_b, av_b, bv_b,
                 gsem_b, asem_b, bsem_b, osem_b, dsem_b,
                 rows_a, av_a, bv_a, gsem_a, asem_a, bsem_a, osem_a)

        half(SEG - 1, rows_a, av_a, bv_a,
             gsem_a, asem_a, bsem_a, osem_a, dsem_a,
             rows_b, av_b, bv_b, gsem_b, asem_b, bsem_b, osem_b)

        pltpu.make_async_copy(rows_b, out_sh.at[dst2_v.at[0]], osem_b).wait()
        pltpu.make_async_copy(rows_a, out_sh.at[dst2_v.at[0]], osem_a).wait()
        pltpu.make_async_copy(ex_v.at[pl.ds(0, CHUNK)],
                              den_sh.at[dst2_v.at[0]], dsem_b).wait()
        pltpu.make_async_copy(ex_v.at[pl.ds(0, CHUNK)],
                              den_sh.at[dst2_v.at[0]], dsem_a).wait()

    plsc.subcore_barrier()

    r0 = sid * ROWS_PT
    pltpu.sync_copy(out_sh.at[pl.ds(r0, ROWS_PT)],
                    outp_hbm.at[cid, pl.ds(r0, ROWS_PT)])
    d0 = sid * DEN_PT
    pltpu.sync_copy(den_sh.at[pl.ds(d0, DEN_PT)],
                    denp_hbm.at[cid, pl.ds(d0, DEN_PT)])


def _fin_body(o_ref, d_ref, b_ref, p_ref, out_ref):
    o = o_ref[0] + o_ref[1]
    dsum = d_ref[0:1, :] + d_ref[1:2, :]
    eye = jnp.eye(128, dtype=jnp.float32)
    cols = [jax.lax.dot_general(eye, dsum[:, 128 * k:128 * (k + 1)],
                                (((1,), (1,)), ((), ())),
                                preferred_element_type=jnp.float32)
            for k in range(_FB // 128)]
    den = jnp.concatenate(cols, axis=0)
    r = o / (den + 1e-16) + b_ref[...]
    out_ref[...] = jnp.where(r >= 0.0, r, p_ref[...] * r)


_FB = 512


def _finalize(outp, denp, bias2, prelu2):
    return pl.pallas_call(
        _fin_body,
        grid=(NPAD // _FB,),
        in_specs=[
            pl.BlockSpec((NC, _FB, CH), lambda i: (0, i, 0)),
            pl.BlockSpec((NC, _FB), lambda i: (0, i)),
            pl.BlockSpec((1, CH), lambda i: (0, 0)),
            pl.BlockSpec((1, CH), lambda i: (0, 0)),
        ],
        out_specs=pl.BlockSpec((_FB, CH), lambda i: (i, 0)),
        out_shape=jax.ShapeDtypeStruct((NPAD, CH), jnp.float32),
    )(outp, denp, bias2, prelu2)


def kernel(x, edge_index, W, a_src, a_dst, bias, prelu_w):
    src = edge_index[0].astype(jnp.int32)
    dst = edge_index[1].astype(jnp.int32)
    a_src_v = a_src.reshape(1, CH).astype(jnp.float32)
    a_dst_v = a_dst.reshape(1, CH).astype(jnp.float32)

    xpad = jnp.pad(x, ((0, NPAD - N), (0, 0)))
    xp, asrc2, adst2, cmax = _project(xpad, W, a_src_v, a_dst_v)
    cvec = jnp.broadcast_to(cmax[0, 0], (L,))

    src3 = src.reshape(NW, NSEG, SEG, CHUNK)
    dst3 = dst.reshape(NW, NSEG, SEG, CHUNK)
    outp, denp = _sc_edges(xp, asrc2.reshape(NPAD), adst2.reshape(NPAD),
                           src3, dst3, cvec)

    out = _finalize(outp, denp, bias.reshape(1, CH), prelu_w.reshape(1, CH))
    return out[:N]
